# baseline (speedup 1.0000x reference)
import numpy as np

import jax
import jax.numpy as jnp
from jax.experimental import pallas as pl
from jax.experimental.pallas import tpu as pltpu

_ALPHA = 0.2
_LAM = float(np.random.RandomState(0).beta(_ALPHA, _ALPHA))


def _mix_kernel(perm_ref, x_ref, xp_ref, l_ref, lp_ref, ox_ref, ol_ref):
    del perm_ref
    ox_ref[...] = _LAM * x_ref[...] + (1.0 - _LAM) * xp_ref[...]
    ol_ref[...] = _LAM * l_ref[...] + (1.0 - _LAM) * lp_ref[...]


def kernel(batch_spectrograms, batch_labels):
    B, C, H, W = batch_spectrograms.shape
    L = batch_labels.shape[1]
    perm = jax.random.permutation(jax.random.key(42), B)
    labels3 = batch_labels[:, None, :]

    grid_spec = pltpu.PrefetchScalarGridSpec(
        num_scalar_prefetch=1,
        grid=(B,),
        in_specs=[
            pl.BlockSpec((1, C, H, W), lambda i, p: (i, 0, 0, 0)),
            pl.BlockSpec((1, C, H, W), lambda i, p: (p[i], 0, 0, 0)),
            pl.BlockSpec((1, 1, L), lambda i, p: (i, 0, 0)),
            pl.BlockSpec((1, 1, L), lambda i, p: (p[i], 0, 0)),
        ],
        out_specs=[
            pl.BlockSpec((1, C, H, W), lambda i, p: (i, 0, 0, 0)),
            pl.BlockSpec((1, 1, L), lambda i, p: (i, 0, 0)),
        ],
    )
    ox, ol = pl.pallas_call(
        _mix_kernel,
        grid_spec=grid_spec,
        out_shape=[
            jax.ShapeDtypeStruct(batch_spectrograms.shape, jnp.float32),
            jax.ShapeDtypeStruct(labels3.shape, jnp.float32),
        ],
    )(perm, batch_spectrograms, batch_spectrograms, labels3, labels3)
    return ox, ol[:, 0, :]


# speedup vs baseline: 1.3180x; 1.3180x over previous
import numpy as np

import jax
import jax.numpy as jnp
from jax.experimental import pallas as pl
from jax.experimental.pallas import tpu as pltpu

_ALPHA = 0.2
_LAM = float(np.random.RandomState(0).beta(_ALPHA, _ALPHA))


def _mix_kernel(perm_ref, x_ref, l_ref, ox_ref, ol_ref):
    i = pl.program_id(0)
    j = perm_ref[i]
    ox_ref[0, 0] = _LAM * x_ref[i, 0] + (1.0 - _LAM) * x_ref[j, 0]
    ol_ref[0, 0] = _LAM * l_ref[i, 0] + (1.0 - _LAM) * l_ref[j, 0]


def kernel(batch_spectrograms, batch_labels):
    B, C, H, W = batch_spectrograms.shape
    L = batch_labels.shape[1]
    perm = jax.random.permutation(jax.random.key(42), B)
    labels3 = batch_labels[:, None, :]

    grid_spec = pltpu.PrefetchScalarGridSpec(
        num_scalar_prefetch=1,
        grid=(B,),
        in_specs=[
            pl.BlockSpec((B, C, H, W), lambda i, p: (0, 0, 0, 0)),
            pl.BlockSpec((B, 1, L), lambda i, p: (0, 0, 0)),
        ],
        out_specs=[
            pl.BlockSpec((1, C, H, W), lambda i, p: (i, 0, 0, 0)),
            pl.BlockSpec((1, 1, L), lambda i, p: (i, 0, 0)),
        ],
    )
    ox, ol = pl.pallas_call(
        _mix_kernel,
        grid_spec=grid_spec,
        out_shape=[
            jax.ShapeDtypeStruct(batch_spectrograms.shape, jnp.float32),
            jax.ShapeDtypeStruct(labels3.shape, jnp.float32),
        ],
    )(perm, batch_spectrograms, labels3)
    return ox, ol[:, 0, :]


# speedup vs baseline: 1.3676x; 1.0377x over previous
import numpy as np

import jax
import jax.numpy as jnp
from jax.experimental import pallas as pl
from jax.experimental.pallas import tpu as pltpu

_ALPHA = 0.2
_LAM = float(np.random.RandomState(0).beta(_ALPHA, _ALPHA))

_NCHUNK = 16


def _mix_kernel(order_ref, po_ref, needed_ref, x_hbm, l_ref, ox_ref, ol_ref,
                buf, sems, waited):
    g = pl.program_id(0)
    nrows = x_hbm.shape[0]
    rpc = nrows // _NCHUNK

    @pl.when(g == 0)
    def _():
        waited[0] = 0
        for c in range(_NCHUNK):
            pltpu.make_async_copy(
                x_hbm.at[pl.ds(c * rpc, rpc)],
                buf.at[pl.ds(c * rpc, rpc)],
                sems.at[c],
            ).start()

    need = needed_ref[g]
    w0 = waited[0]
    for c in range(_NCHUNK):
        @pl.when(jnp.logical_and(c >= w0, c <= need))
        def _(c=c):
            pltpu.make_async_copy(
                x_hbm.at[pl.ds(c * rpc, rpc)],
                buf.at[pl.ds(c * rpc, rpc)],
                sems.at[c],
            ).wait()
    waited[0] = jnp.maximum(w0, need + 1)

    i = order_ref[g]
    j = po_ref[g]
    ox_ref[0, 0] = _LAM * buf[i, 0] + (1.0 - _LAM) * buf[j, 0]
    ol_ref[0, 0] = _LAM * l_ref[i, 0] + (1.0 - _LAM) * l_ref[j, 0]


def kernel(batch_spectrograms, batch_labels):
    B, C, H, W = batch_spectrograms.shape
    L = batch_labels.shape[1]
    rpc = B // _NCHUNK
    perm = jax.random.permutation(jax.random.key(42), B)

    rows = jnp.arange(B, dtype=jnp.int32)
    last_chunk = jnp.maximum(rows // rpc, perm.astype(jnp.int32) // rpc)
    order = jnp.argsort(last_chunk, stable=True).astype(jnp.int32)
    po = perm.astype(jnp.int32)[order]
    needed = last_chunk[order]

    labels3 = batch_labels[:, None, :]

    grid_spec = pltpu.PrefetchScalarGridSpec(
        num_scalar_prefetch=3,
        grid=(B,),
        in_specs=[
            pl.BlockSpec(memory_space=pl.ANY),
            pl.BlockSpec((B, 1, L), lambda g, o, p, n: (0, 0, 0)),
        ],
        out_specs=[
            pl.BlockSpec((1, C, H, W), lambda g, o, p, n: (o[g], 0, 0, 0)),
            pl.BlockSpec((1, 1, L), lambda g, o, p, n: (o[g], 0, 0)),
        ],
        scratch_shapes=[
            pltpu.VMEM((B, C, H, W), jnp.float32),
            pltpu.SemaphoreType.DMA((_NCHUNK,)),
            pltpu.SMEM((1,), jnp.int32),
        ],
    )
    ox, ol = pl.pallas_call(
        _mix_kernel,
        grid_spec=grid_spec,
        out_shape=[
            jax.ShapeDtypeStruct(batch_spectrograms.shape, jnp.float32),
            jax.ShapeDtypeStruct(labels3.shape, jnp.float32),
        ],
    )(order, po, needed, batch_spectrograms, labels3)
    return ox, ol[:, 0, :]
